# baseline (speedup 1.0000x reference)
import functools

import jax
import jax.numpy as jnp
from jax import lax
from jax.experimental import pallas as pl
from jax.experimental.pallas import tpu as pltpu
from jax.experimental.pallas import tpu_sc as plsc

_NC = 2
_NS = 16
_NW = _NC * _NS
_CB = 128
_K = 2
_SB = _K * _CB
_L = 16


@functools.lru_cache(maxsize=None)
def _gather_call(t_dim, b_dim, d):
    sc_per_t = b_dim // _SB
    sc_total = t_dim * sc_per_t
    sc_per_w = sc_total // _NW
    pairs = sc_per_w // 2
    mesh = plsc.VectorSubcoreMesh(core_axis_name="c", subcore_axis_name="s")

    @functools.partial(
        pl.kernel,
        mesh=mesh,
        out_type=jax.ShapeDtypeStruct((t_dim, d, b_dim), jnp.float32),
        compiler_params=pltpu.CompilerParams(
            use_tc_tiling_on_sc=False, needs_layout_passes=False),
        scratch_types=[
            pltpu.VMEM((sc_per_w * _K, _CB), jnp.int32),
            pltpu.VMEM((_SB, d), jnp.float32),
            pltpu.VMEM((_SB, d), jnp.float32),
            pltpu.VMEM((d, _SB), jnp.float32),
            pltpu.VMEM((d, _SB), jnp.float32),
            pltpu.SemaphoreType.DMA,
            pltpu.SemaphoreType.DMA,
            pltpu.SemaphoreType.DMA,
            pltpu.SemaphoreType.DMA,
        ],
    )
    def k(idx_hbm, table_hbm, out_hbm, idx_v, rows0, rows1, tb0, tb1,
          gsem0, gsem1, ssem0, ssem1):
        wid = lax.axis_index("s") * _NC + lax.axis_index("c")
        c0 = wid * sc_per_w * _K
        pltpu.sync_copy(idx_hbm.at[pl.ds(c0, sc_per_w * _K)], idx_v)
        rows = (rows0, rows1)
        tbs = (tb0, tb1)
        gsems = (gsem0, gsem1)
        ssems = (ssem0, ssem1)
        row_ids = [lax.iota(jnp.int32, _L) + (_L * g) for g in range(_SB // _L)]

        def out_slice(u):
            ug = c0 // _K + u
            t = ug // sc_per_t
            sb = lax.rem(ug, sc_per_t)
            return out_hbm.at[t, :, pl.ds(sb * _SB, _SB)]

        def fire_gathers(b, u):
            for kk in range(_K):
                pltpu.async_copy(
                    table_hbm.at[idx_v.at[u * _K + kk]],
                    rows[b].at[pl.ds(kk * _CB, _CB)],
                    gsems[b])

        def wait_gathers(b, u):
            pltpu.make_async_copy(
                table_hbm.at[pl.ds(0, _SB)], rows[b], gsems[b]).wait()

        def transpose(b):
            src, dst = rows[b], tbs[b]

            def col(c, carry):
                col_ids = jnp.full((_L,), c, jnp.int32)
                for g in range(_SB // _L):
                    v = plsc.load_gather(src, [row_ids[g], col_ids])
                    dst[c, pl.ds(_L * g, _L)] = v
                return carry

            lax.fori_loop(0, d, col, 0, unroll=4)

        def fire_store(b, u):
            pltpu.make_async_copy(tbs[b], out_slice(u), ssems[b]).start()

        def wait_store(b, u):
            pltpu.make_async_copy(tbs[b], out_slice(u), ssems[b]).wait()

        fire_gathers(0, 0)
        fire_gathers(1, 1)

        def body(p, carry):
            for b in range(2):
                u = 2 * p + b
                wait_gathers(b, u)

                @pl.when(p > 0)
                def _():
                    wait_store(b, u)

                transpose(b)
                fire_store(b, u)

                @pl.when(u + 2 < sc_per_w)
                def _():
                    fire_gathers(b, u + 2)

            return carry

        lax.fori_loop(0, pairs, body, 0)
        wait_store(0, sc_per_w - 2)
        wait_store(1, sc_per_w - 1)

    return k


def kernel(input, embedding_weight):
    b, h = input.shape
    v, d = embedding_weight.shape
    idx2d = input.T.reshape((b * h) // _CB, _CB)
    out3 = _gather_call(h, b, d)(idx2d, embedding_weight)
    return out3.transpose(2, 0, 1)


# speedup vs baseline: 1.3935x; 1.3935x over previous
import functools

import jax
import jax.numpy as jnp
from jax import lax
from jax.experimental import pallas as pl
from jax.experimental.pallas import tpu as pltpu
from jax.experimental.pallas import tpu_sc as plsc

_NC = 2
_NS = 16
_NW = _NC * _NS
_CB = 128
_K = 2
_SB = _K * _CB
_L = 16


@functools.lru_cache(maxsize=None)
def _gather_call(t_dim, b_dim, d):
    sc_per_t = b_dim // _SB
    sc_total = t_dim * sc_per_t
    sc_per_w = sc_total // _NW
    pairs = sc_per_w // 2
    mesh = plsc.VectorSubcoreMesh(core_axis_name="c", subcore_axis_name="s")

    @functools.partial(
        pl.kernel,
        mesh=mesh,
        out_type=jax.ShapeDtypeStruct((t_dim, d, b_dim), jnp.float32),
        compiler_params=pltpu.CompilerParams(
            use_tc_tiling_on_sc=False, needs_layout_passes=False),
        scratch_types=[
            pltpu.VMEM((sc_per_w * _K, _CB), jnp.int32),
            pltpu.VMEM((_SB, d), jnp.float32),
            pltpu.VMEM((_SB, d), jnp.float32),
            pltpu.VMEM((d, _SB), jnp.float32),
            pltpu.VMEM((d, _SB), jnp.float32),
            pltpu.SemaphoreType.DMA,
            pltpu.SemaphoreType.DMA,
            pltpu.SemaphoreType.DMA,
            pltpu.SemaphoreType.DMA,
        ],
    )
    def k(idx_hbm, table_hbm, out_hbm, idx_v, rows0, rows1, tb0, tb1,
          gsem0, gsem1, ssem0, ssem1):
        wid = lax.axis_index("s") * _NC + lax.axis_index("c")
        c0 = wid * sc_per_w * _K
        pltpu.sync_copy(idx_hbm.at[pl.ds(c0, sc_per_w * _K)], idx_v)
        rows = (rows0, rows1)
        tbs = (tb0, tb1)
        gsems = (gsem0, gsem1)
        ssems = (ssem0, ssem1)
        row_ids = [lax.iota(jnp.int32, _L) + (_L * g) for g in range(_SB // _L)]

        def out_slice(u):
            ug = c0 // _K + u
            t = ug // sc_per_t
            sb = lax.rem(ug, sc_per_t)
            return out_hbm.at[t, :, pl.ds(sb * _SB, _SB)]

        def fire_gathers(b, u):
            for kk in range(_K):
                pltpu.async_copy(
                    table_hbm.at[idx_v.at[u * _K + kk]],
                    rows[b].at[pl.ds(kk * _CB, _CB)],
                    gsems[b])

        def wait_gathers(b, u):
            pltpu.make_async_copy(
                table_hbm.at[pl.ds(0, _SB)], rows[b], gsems[b]).wait()

        def transpose(b):
            src, dst = rows[b], tbs[b]

            @plsc.parallel_loop(0, d, unroll=4)
            def col(c):
                col_ids = jnp.full((_L,), c, jnp.int32)
                for g in range(_SB // _L):
                    v = plsc.load_gather(src, [row_ids[g], col_ids])
                    dst[c, pl.ds(_L * g, _L)] = v

        def fire_store(b, u):
            pltpu.make_async_copy(tbs[b], out_slice(u), ssems[b]).start()

        def wait_store(b, u):
            pltpu.make_async_copy(tbs[b], out_slice(u), ssems[b]).wait()

        fire_gathers(0, 0)
        fire_gathers(1, 1)

        def body(p, carry):
            for b in range(2):
                u = 2 * p + b
                wait_gathers(b, u)

                @pl.when(p > 0)
                def _():
                    wait_store(b, u)

                transpose(b)
                fire_store(b, u)

                @pl.when(u + 2 < sc_per_w)
                def _():
                    fire_gathers(b, u + 2)

            return carry

        lax.fori_loop(0, pairs, body, 0)
        wait_store(0, sc_per_w - 2)
        wait_store(1, sc_per_w - 1)

    return k


def kernel(input, embedding_weight):
    b, h = input.shape
    v, d = embedding_weight.shape
    idx2d = input.T.reshape((b * h) // _CB, _CB)
    out3 = _gather_call(h, b, d)(idx2d, embedding_weight)
    return out3.transpose(2, 0, 1)


# speedup vs baseline: 2.0984x; 1.5059x over previous
import functools

import jax
import jax.numpy as jnp
from jax import lax
from jax.experimental import pallas as pl
from jax.experimental.pallas import tpu as pltpu
from jax.experimental.pallas import tpu_sc as plsc

_NC = 2
_NS = 16
_NW = _NC * _NS
_CB = 128
_K = 2
_SB = _K * _CB
_L = 16


@functools.lru_cache(maxsize=None)
def _gather_call(t_dim, b_dim, d):
    sc_per_t = b_dim // _SB
    sc_total = t_dim * sc_per_t
    sc_per_w = sc_total // _NW
    pairs = sc_per_w // 2
    mesh = plsc.VectorSubcoreMesh(core_axis_name="c", subcore_axis_name="s")

    @functools.partial(
        pl.kernel,
        mesh=mesh,
        out_type=jax.ShapeDtypeStruct((t_dim, d, b_dim), jnp.float32),
        compiler_params=pltpu.CompilerParams(
            use_tc_tiling_on_sc=False, needs_layout_passes=False),
        scratch_types=[
            pltpu.VMEM((sc_per_w * _K, _CB), jnp.int32),
            pltpu.VMEM((_SB, d), jnp.float32),
            pltpu.VMEM((_SB, d), jnp.float32),
            pltpu.VMEM((d, _SB + 1), jnp.float32),
            pltpu.VMEM((d, _SB + 1), jnp.float32),
            pltpu.SemaphoreType.DMA,
            pltpu.SemaphoreType.DMA,
            pltpu.SemaphoreType.DMA,
            pltpu.SemaphoreType.DMA,
        ],
    )
    def k(idx_hbm, table_hbm, out_hbm, idx_v, rows0, rows1, tb0, tb1,
          gsem0, gsem1, ssem0, ssem1):
        wid = lax.axis_index("s") * _NC + lax.axis_index("c")
        c0 = wid * sc_per_w * _K
        pltpu.sync_copy(idx_hbm.at[pl.ds(c0, sc_per_w * _K)], idx_v)
        rows = (rows0, rows1)
        tbs = (tb0, tb1)
        gsems = (gsem0, gsem1)
        ssems = (ssem0, ssem1)
        col_ids = [lax.iota(jnp.int32, _L) + (_L * g) for g in range(d // _L)]

        def out_slice(u):
            ug = c0 // _K + u
            t = ug // sc_per_t
            sb = lax.rem(ug, sc_per_t)
            return out_hbm.at[t, :, pl.ds(sb * _SB, _SB)]

        def fire_gathers(b, u):
            for kk in range(_K):
                pltpu.async_copy(
                    table_hbm.at[idx_v.at[u * _K + kk]],
                    rows[b].at[pl.ds(kk * _CB, _CB)],
                    gsems[b])

        def wait_gathers(b, u):
            pltpu.make_async_copy(
                table_hbm.at[pl.ds(0, _SB)], rows[b], gsems[b]).wait()

        def transpose(b):
            src, dst = rows[b], tbs[b]

            @plsc.parallel_loop(0, _SB, unroll=4)
            def rowfn(r):
                r_ids = jnp.full((_L,), r, jnp.int32)
                for g in range(d // _L):
                    v = src[r, pl.ds(_L * g, _L)]
                    plsc.store_scatter(dst, [col_ids[g], r_ids], v)

        def fire_store(b, u):
            pltpu.make_async_copy(
                tbs[b].at[:, pl.ds(0, _SB)], out_slice(u), ssems[b]).start()

        def wait_store(b, u):
            pltpu.make_async_copy(
                tbs[b].at[:, pl.ds(0, _SB)], out_slice(u), ssems[b]).wait()

        fire_gathers(0, 0)
        fire_gathers(1, 1)

        def body(p, carry):
            for b in range(2):
                u = 2 * p + b
                wait_gathers(b, u)

                @pl.when(p > 0)
                def _():
                    wait_store(b, u)

                transpose(b)
                fire_store(b, u)

                @pl.when(u + 2 < sc_per_w)
                def _():
                    fire_gathers(b, u + 2)

            return carry

        lax.fori_loop(0, pairs, body, 0)
        wait_store(0, sc_per_w - 2)
        wait_store(1, sc_per_w - 1)

    return k


def kernel(input, embedding_weight):
    b, h = input.shape
    v, d = embedding_weight.shape
    idx2d = input.T.reshape((b * h) // _CB, _CB)
    out3 = _gather_call(h, b, d)(idx2d, embedding_weight)
    return out3.transpose(2, 0, 1)
